# baseline (speedup 1.0000x reference)
import functools

import jax
import jax.numpy as jnp
from jax import lax
from jax.experimental import pallas as pl
from jax.experimental.pallas import tpu as pltpu
from jax.experimental.pallas import tpu_sc as plsc

POS_BND = 11
RPE_NUM = 2 * POS_BND + 1
NUM_HEADS = 16
NV = 12
NFUSED = NV * NV * NV

NC = 2
NS = 16
NW = NC * NS
L = 16

B = 1024
HW = 48
JT = 6
NPOS = B * HW * HW
BLK = 2048
NBLKS = NPOS // BLK
BLKS_PER_W = NBLKS // NW
O_H_STRIDE = NPOS
N_CHUNKS = BLK // L


def _sc_call():
    mesh = plsc.VectorSubcoreMesh(core_axis_name="c", subcore_axis_name="s")

    @functools.partial(
        pl.kernel,
        mesh=mesh,
        out_type=jax.ShapeDtypeStruct((NUM_HEADS * NPOS,), jnp.float32),
        scratch_types=[
            pltpu.VMEM((3 * RPE_NUM * NUM_HEADS,), jnp.float32),
            pltpu.VMEM((NUM_HEADS * NFUSED,), jnp.float32),
            pltpu.VMEM((3 * BLK,), jnp.int32),
            pltpu.VMEM((3 * BLK,), jnp.int32),
            pltpu.VMEM((NUM_HEADS * BLK,), jnp.float32),
            pltpu.VMEM((NUM_HEADS * BLK,), jnp.float32),
            pltpu.SemaphoreType.DMA,
            pltpu.SemaphoreType.DMA,
            pltpu.SemaphoreType.DMA,
            pltpu.SemaphoreType.DMA,
        ],
        compiler_params=pltpu.CompilerParams(needs_layout_passes=False),
    )
    def sc_call(coord_hbm, tflat_hbm, out_hbm, tflat_v, fused_v,
                cb0, cb1, ob0, ob1, cs0, cs1, os0, os1):
        wid = lax.axis_index("s") * NC + lax.axis_index("c")
        s0 = wid * BLKS_PER_W
        cbufs = ((cb0, cs0), (cb1, cs1))
        obufs = ((ob0, os0), (ob1, os1))

        iota = lax.iota(jnp.int32, L)

        def fire_coord(s, buf, sem):
            ig = s // 24
            off = (s % 24) * BLK
            base = ig * (3 * 49152) + off
            for c in range(3):
                pltpu.async_copy(
                    coord_hbm.at[pl.ds(base + c * 49152, BLK)],
                    buf.at[pl.ds(c * BLK, BLK)],
                    sem,
                )

        def wait_coord(buf, sem):
            pltpu.make_async_copy(
                coord_hbm.at[pl.ds(0, 3 * BLK)], buf, sem
            ).wait()

        fire_coord(s0, cb0, cs0)

        pltpu.sync_copy(tflat_hbm, tflat_v)

        def build(tc, carry):
            t16 = tc * L + iota
            c0 = t16 // (NV * NV)
            r = t16 % (NV * NV)
            c1 = r // NV
            c2 = r % NV
            for h in range(NUM_HEADS):
                base = h * (3 * RPE_NUM)
                g = (
                    plsc.load_gather(tflat_v, [base + POS_BND + c0])
                    + plsc.load_gather(tflat_v, [base + RPE_NUM + POS_BND + c1])
                    + plsc.load_gather(tflat_v, [base + 2 * RPE_NUM + POS_BND + c2])
                )
                fused_v[pl.ds(h * NFUSED + tc * L, L)] = g
            return carry

        lax.fori_loop(0, NFUSED // L, build, 0)

        def block_pass(cbuf, obuf):
            @plsc.parallel_loop(0, BLK, step=L, unroll=4)
            def chunk(o):
                c0 = cbuf[pl.ds(o, L)]
                c1 = cbuf[pl.ds(BLK + o, L)]
                c2 = cbuf[pl.ds(2 * BLK + o, L)]
                c0 = jnp.minimum(jnp.maximum(c0, 0), NV - 1)
                c1 = jnp.minimum(jnp.maximum(c1, 0), NV - 1)
                c2 = jnp.minimum(jnp.maximum(c2, 0), NV - 1)
                t = c0 * (NV * NV) + c1 * NV + c2
                for h in range(NUM_HEADS):
                    g = plsc.load_gather(fused_v, [t + h * NFUSED])
                    obuf[pl.ds(h * BLK + o, L)] = g

        def pair(sp, carry):
            for k in range(2):
                s = s0 + sp * 2 + k
                cbuf, csem = cbufs[k]
                obuf, osem = obufs[k]
                wait_coord(cbuf, csem)
                @pl.when(sp * 2 + k >= 2)
                def _():
                    pltpu.make_async_copy(
                        obuf, out_hbm.at[pl.ds(0, NUM_HEADS * BLK)], osem
                    ).wait()
                block_pass(cbuf, obuf)
                @pl.when(sp * 2 + k + 2 < BLKS_PER_W)
                def _():
                    fire_coord(s + 2, cbuf, csem)

                pos = (s // 24) * 49152 + (s % 24) * BLK
                for h in range(NUM_HEADS):
                    pltpu.async_copy(
                        obuf.at[pl.ds(h * BLK, BLK)],
                        out_hbm.at[pl.ds(pos + h * O_H_STRIDE, BLK)],
                        osem,
                    )
            return carry

        fire_coord(s0 + 1, cb1, cs1)

        lax.fori_loop(0, BLKS_PER_W // 2, pair, 0)

        for k in range(2):
            obuf, osem = obufs[k]
            pltpu.make_async_copy(
                obuf, out_hbm.at[pl.ds(0, NUM_HEADS * BLK)], osem
            ).wait()

    return sc_call


@jax.jit
def kernel(coord, table):
    coordf = (
        coord.reshape(8, 128, HW, JT, 8, 3)
        .transpose(2, 5, 3, 0, 4, 1)
        .reshape(-1)
    )
    tflat = table.T.reshape(-1)
    out = _sc_call()(coordf, tflat)
    return (
        out.reshape(NUM_HEADS, HW, JT, 8, 8, 128)
        .transpose(3, 5, 0, 1, 2, 4)
        .reshape(B, NUM_HEADS, HW, HW)
    )


# speedup vs baseline: 1.0922x; 1.0922x over previous
import functools

import jax
import jax.numpy as jnp
from jax import lax
from jax.experimental import pallas as pl
from jax.experimental.pallas import tpu as pltpu
from jax.experimental.pallas import tpu_sc as plsc

POS_BND = 11
RPE_NUM = 2 * POS_BND + 1
NUM_HEADS = 16
NV = 12
NFUSED = NV * NV * NV

NC = 2
NS = 16
NW = NC * NS
L = 16

B = 1024
HW = 48
JT = 6
NPOS = B * HW * HW
BLK = 2048
NBLKS = NPOS // BLK
BLKS_PER_W = NBLKS // NW
O_H_STRIDE = NPOS
N_CHUNKS = BLK // L


def _sc_call():
    mesh = plsc.VectorSubcoreMesh(core_axis_name="c", subcore_axis_name="s")

    @functools.partial(
        pl.kernel,
        mesh=mesh,
        out_type=jax.ShapeDtypeStruct((NUM_HEADS * NPOS,), jnp.float32),
        scratch_types=[
            pltpu.VMEM((3 * RPE_NUM * NUM_HEADS,), jnp.float32),
            pltpu.VMEM((NUM_HEADS * NFUSED,), jnp.float32),
            pltpu.VMEM((3 * BLK,), jnp.int32),
            pltpu.VMEM((3 * BLK,), jnp.int32),
            pltpu.VMEM((NUM_HEADS * BLK,), jnp.float32),
            pltpu.VMEM((NUM_HEADS * BLK,), jnp.float32),
            pltpu.SemaphoreType.DMA,
            pltpu.SemaphoreType.DMA,
            pltpu.SemaphoreType.DMA,
            pltpu.SemaphoreType.DMA,
        ],
        compiler_params=pltpu.CompilerParams(needs_layout_passes=False),
    )
    def sc_call(coord_hbm, tflat_hbm, out_hbm, tflat_v, fused_v,
                cb0, cb1, ob0, ob1, cs0, cs1, os0, os1):
        wid = lax.axis_index("s") * NC + lax.axis_index("c")
        s0 = wid * BLKS_PER_W
        cbufs = ((cb0, cs0), (cb1, cs1))
        obufs = ((ob0, os0), (ob1, os1))

        iota = lax.iota(jnp.int32, L)

        def fire_coord(s, buf, sem):
            ig = s // 24
            off = (s % 24) * BLK
            base = ig * (3 * 49152) + off
            for c in range(3):
                pltpu.async_copy(
                    coord_hbm.at[pl.ds(base + c * 49152, BLK)],
                    buf.at[pl.ds(c * BLK, BLK)],
                    sem,
                )

        def wait_coord(buf, sem):
            pltpu.make_async_copy(
                coord_hbm.at[pl.ds(0, 3 * BLK)], buf, sem
            ).wait()

        fire_coord(s0, cb0, cs0)

        pltpu.sync_copy(tflat_hbm, tflat_v)

        @plsc.parallel_loop(0, NFUSED, step=L, unroll=2)
        def build(tb):
            t16 = tb + iota
            c0 = t16 // (NV * NV)
            r = t16 % (NV * NV)
            c1 = r // NV
            c2 = r % NV
            for h in range(NUM_HEADS):
                base = h * (3 * RPE_NUM)
                g = (
                    plsc.load_gather(tflat_v, [base + POS_BND + c0])
                    + plsc.load_gather(tflat_v, [base + RPE_NUM + POS_BND + c1])
                    + plsc.load_gather(tflat_v, [base + 2 * RPE_NUM + POS_BND + c2])
                )
                fused_v[pl.ds(h * NFUSED + tb, L)] = g

        def block_pass(cbuf, obuf):
            @plsc.parallel_loop(0, BLK, step=L, unroll=2)
            def chunk(o):
                c0 = cbuf[pl.ds(o, L)]
                c1 = cbuf[pl.ds(BLK + o, L)]
                c2 = cbuf[pl.ds(2 * BLK + o, L)]
                c0 = jnp.minimum(jnp.maximum(c0, 0), NV - 1)
                c1 = jnp.minimum(jnp.maximum(c1, 0), NV - 1)
                c2 = jnp.minimum(jnp.maximum(c2, 0), NV - 1)
                t = c0 * (NV * NV) + c1 * NV + c2
                for h in range(NUM_HEADS):
                    g = plsc.load_gather(fused_v, [t + h * NFUSED])
                    obuf[pl.ds(h * BLK + o, L)] = g

        def pair(sp, carry):
            for k in range(2):
                s = s0 + sp * 2 + k
                cbuf, csem = cbufs[k]
                obuf, osem = obufs[k]
                wait_coord(cbuf, csem)
                @pl.when(sp * 2 + k >= 2)
                def _():
                    pltpu.make_async_copy(
                        obuf, out_hbm.at[pl.ds(0, NUM_HEADS * BLK)], osem
                    ).wait()
                block_pass(cbuf, obuf)
                @pl.when(sp * 2 + k + 2 < BLKS_PER_W)
                def _():
                    fire_coord(s + 2, cbuf, csem)

                pos = (s // 24) * 49152 + (s % 24) * BLK
                for h in range(NUM_HEADS):
                    pltpu.async_copy(
                        obuf.at[pl.ds(h * BLK, BLK)],
                        out_hbm.at[pl.ds(pos + h * O_H_STRIDE, BLK)],
                        osem,
                    )
            return carry

        fire_coord(s0 + 1, cb1, cs1)

        lax.fori_loop(0, BLKS_PER_W // 2, pair, 0)

        for k in range(2):
            obuf, osem = obufs[k]
            pltpu.make_async_copy(
                obuf, out_hbm.at[pl.ds(0, NUM_HEADS * BLK)], osem
            ).wait()

    return sc_call


@jax.jit
def kernel(coord, table):
    coordf = (
        coord.reshape(8, 128, HW, JT, 8, 3)
        .transpose(2, 5, 3, 0, 4, 1)
        .reshape(-1)
    )
    tflat = table.T.reshape(-1)
    out = _sc_call()(coordf, tflat)
    return (
        out.reshape(NUM_HEADS, HW, JT, 8, 8, 128)
        .transpose(3, 5, 0, 1, 2, 4)
        .reshape(B, NUM_HEADS, HW, HW)
    )
